# Initial kernel scaffold; baseline (speedup 1.0000x reference)
import jax
import jax.numpy as jnp
from jax.experimental import pallas as pl


def kernel(input_ids, emb_table, proj_table):
    raise NotImplementedError("write your pallas kernel here")


# speedup vs baseline: 1.5033x; 1.5033x over previous
import functools

import jax
import jax.numpy as jnp
import numpy as np
from jax import lax
from jax.experimental import pallas as pl
from jax.experimental.pallas import tpu as pltpu
from jax.experimental.pallas import tpu_sc as plsc

NUM_EMB = 100000
HIDDEN = 128
PROJ_W = 32
OUT_W = 512
LANES = 16

NC, NS = 2, 16
NW = NC * NS

CHUNK = 64


def _sc_kernel(num_tokens):
    assert num_tokens % (NW * CHUNK) == 0
    per_w = num_tokens // NW
    n_chunks = per_w // CHUNK
    mesh = plsc.VectorSubcoreMesh(core_axis_name="c", subcore_axis_name="s")

    @functools.partial(
        pl.kernel,
        out_type=jax.ShapeDtypeStruct((num_tokens, OUT_W), jnp.float32),
        mesh=mesh,
        compiler_params=pltpu.CompilerParams(
            use_tc_tiling_on_sc=False, needs_layout_passes=False),
        scratch_types=[
            pltpu.VMEM((n_chunks, CHUNK), jnp.int32),
            pltpu.VMEM((CHUNK, HIDDEN), jnp.float32),
            pltpu.VMEM((CHUNK, PROJ_W), jnp.float32),
            pltpu.VMEM((CHUNK, OUT_W), jnp.float32),
            pltpu.SemaphoreType.DMA,
            pltpu.SemaphoreType.DMA,
            pltpu.SemaphoreType.DMA,
        ],
    )
    def kern(ids_hbm, emb_hbm, proj_hbm, out_hbm,
             ids_v, emb_v, proj_v, out_v, sem_e, sem_p, sem_o):
        wid = lax.axis_index("s") * NC + lax.axis_index("c")
        base = wid * per_w
        pltpu.sync_copy(ids_hbm.at[wid], ids_v)

        quarter = lax.shift_right_logical(lax.iota(jnp.int32, 16), 2)

        def chunk_body(g, _):
            ce = pltpu.async_copy(emb_hbm.at[ids_v.at[g]], emb_v, sem_e)
            cp = pltpu.async_copy(proj_hbm.at[ids_v.at[g]], proj_v, sem_p)
            ce.wait()
            cp.wait()

            def token_body(t, _):
                tsplat = jnp.full((16,), t, jnp.int32)
                p0 = proj_v[t, pl.ds(0, 16)]
                p1 = proj_v[t, pl.ds(16, 16)]
                for j in range(OUT_W // LANES):
                    e = plsc.load_gather(emb_v, [tsplat, quarter + 4 * j])
                    p = p0 if j % 2 == 0 else p1
                    out_v[t, pl.ds(LANES * j, LANES)] = e * p
                return 0

            lax.fori_loop(0, CHUNK, token_body, 0)
            co = pltpu.async_copy(
                out_v, out_hbm.at[pl.ds(base + g * CHUNK, CHUNK)], sem_o)
            co.wait()
            return 0

        lax.fori_loop(0, n_chunks, chunk_body, 0)

    return kern


def kernel(input_ids, emb_table, proj_table):
    bs, l = input_ids.shape
    num_tokens = bs * l
    ids = input_ids.reshape(-1).astype(jnp.int32)
    per_w = num_tokens // NW
    ids3 = ids.reshape(NW, per_w // CHUNK, CHUNK)
    out = _sc_kernel(num_tokens)(ids3, emb_table, proj_table)
    return out.reshape(bs, l, OUT_W)


# speedup vs baseline: 1.6934x; 1.1265x over previous
import functools

import jax
import jax.numpy as jnp
import numpy as np
from jax import lax
from jax.experimental import pallas as pl
from jax.experimental.pallas import tpu as pltpu
from jax.experimental.pallas import tpu_sc as plsc

NUM_EMB = 100000
HIDDEN = 128
PROJ_W = 32
OUT_W = 512
LANES = 16

NC, NS = 2, 16
NW = NC * NS

CHUNK = 64


def _sc_kernel(num_tokens):
    assert num_tokens % (NW * CHUNK) == 0
    per_w = num_tokens // NW
    n_chunks = per_w // CHUNK
    assert n_chunks % 2 == 0
    mesh = plsc.VectorSubcoreMesh(core_axis_name="c", subcore_axis_name="s")

    @functools.partial(
        pl.kernel,
        out_type=jax.ShapeDtypeStruct((num_tokens, OUT_W), jnp.float32),
        mesh=mesh,
        compiler_params=pltpu.CompilerParams(
            use_tc_tiling_on_sc=False, needs_layout_passes=False),
        scratch_types=[
            pltpu.VMEM((n_chunks, CHUNK), jnp.int32),
            [pltpu.VMEM((CHUNK, HIDDEN), jnp.float32) for _ in range(2)],
            [pltpu.VMEM((CHUNK, PROJ_W), jnp.float32) for _ in range(2)],
            [pltpu.VMEM((CHUNK, OUT_W), jnp.float32) for _ in range(2)],
            [pltpu.SemaphoreType.DMA for _ in range(2)],
            [pltpu.SemaphoreType.DMA for _ in range(2)],
            [pltpu.SemaphoreType.DMA for _ in range(2)],
        ],
    )
    def kern(ids_hbm, emb_hbm, proj_hbm, out_hbm,
             ids_v, emb_v, proj_v, out_v, sem_e, sem_p, sem_o):
        wid = lax.axis_index("s") * NC + lax.axis_index("c")
        base = wid * per_w
        pltpu.sync_copy(ids_hbm.at[wid], ids_v)

        quarter = lax.shift_right_logical(lax.iota(jnp.int32, 16), 2)

        def fire_gather(g, b):
            pltpu.async_copy(emb_hbm.at[ids_v.at[g]], emb_v[b], sem_e[b])
            pltpu.async_copy(proj_hbm.at[ids_v.at[g]], proj_v[b], sem_p[b])

        def wait_gather(g, b):
            pltpu.make_async_copy(
                emb_hbm.at[ids_v.at[g]], emb_v[b], sem_e[b]).wait()
            pltpu.make_async_copy(
                proj_hbm.at[ids_v.at[g]], proj_v[b], sem_p[b]).wait()

        def fire_store(g, b):
            pltpu.async_copy(
                out_v[b], out_hbm.at[pl.ds(base + g * CHUNK, CHUNK)], sem_o[b])

        def wait_store(g, b):
            pltpu.make_async_copy(
                out_v[b], out_hbm.at[pl.ds(base + g * CHUNK, CHUNK)],
                sem_o[b]).wait()

        def compute(b):
            ev, pv, ov = emb_v[b], proj_v[b], out_v[b]

            def token_body(t, _):
                tsplat = jnp.full((16,), t, jnp.int32)
                p0 = pv[t, pl.ds(0, 16)]
                p1 = pv[t, pl.ds(16, 16)]
                for j in range(OUT_W // LANES):
                    e = plsc.load_gather(ev, [tsplat, quarter + 4 * j])
                    p = p0 if j % 2 == 0 else p1
                    ov[t, pl.ds(LANES * j, LANES)] = e * p
                return 0

            lax.fori_loop(0, CHUNK, token_body, 0)

        fire_gather(0, 0)
        fire_gather(1, 1)

        def pair_body(i, _):
            for b in range(2):
                g = 2 * i + b

                @pl.when(i > 0)
                def _():
                    wait_store(g - 2, b)

                wait_gather(g, b)
                compute(b)
                fire_store(g, b)

                @pl.when(g + 2 < n_chunks)
                def _():
                    fire_gather(g + 2, b)
            return 0

        lax.fori_loop(0, n_chunks // 2, pair_body, 0)
        wait_store(n_chunks - 2, 0)
        wait_store(n_chunks - 1, 1)

    return kern


def kernel(input_ids, emb_table, proj_table):
    bs, l = input_ids.shape
    num_tokens = bs * l
    ids = input_ids.reshape(-1).astype(jnp.int32)
    per_w = num_tokens // NW
    ids3 = ids.reshape(NW, per_w // CHUNK, CHUNK)
    out = _sc_kernel(num_tokens)(ids3, emb_table, proj_table)
    return out.reshape(bs, l, OUT_W)


# speedup vs baseline: 2.0897x; 1.2340x over previous
import functools

import jax
import jax.numpy as jnp
import numpy as np
from jax import lax
from jax.experimental import pallas as pl
from jax.experimental.pallas import tpu as pltpu
from jax.experimental.pallas import tpu_sc as plsc

NUM_EMB = 100000
HIDDEN = 128
PROJ_W = 32
OUT_W = 512
LANES = 16

NC, NS = 2, 16
NW = NC * NS

CHUNK = 64


def _sc_kernel(num_tokens):
    assert num_tokens % (NW * CHUNK) == 0
    per_w = num_tokens // NW
    n_chunks = per_w // CHUNK
    assert n_chunks % 2 == 0
    mesh = plsc.VectorSubcoreMesh(core_axis_name="c", subcore_axis_name="s")

    @functools.partial(
        pl.kernel,
        out_type=jax.ShapeDtypeStruct((num_tokens, OUT_W), jnp.float32),
        mesh=mesh,
        compiler_params=pltpu.CompilerParams(
            use_tc_tiling_on_sc=False, needs_layout_passes=False),
        scratch_types=[
            pltpu.VMEM((n_chunks, CHUNK), jnp.int32),
            [pltpu.VMEM((CHUNK, HIDDEN), jnp.float32) for _ in range(2)],
            [pltpu.VMEM((CHUNK, PROJ_W), jnp.float32) for _ in range(2)],
            [pltpu.VMEM((CHUNK, OUT_W), jnp.float32) for _ in range(2)],
            [pltpu.SemaphoreType.DMA for _ in range(2)],
            [pltpu.SemaphoreType.DMA for _ in range(2)],
            [pltpu.SemaphoreType.DMA for _ in range(2)],
        ],
    )
    def kern(ids_hbm, emb_hbm, proj_hbm, out_hbm,
             ids_v, emb_v, proj_v, out_v, sem_e, sem_p, sem_o):
        wid = lax.axis_index("s") * NC + lax.axis_index("c")
        base = wid * per_w
        pltpu.sync_copy(ids_hbm.at[wid], ids_v)

        quarter = lax.shift_right_logical(lax.iota(jnp.int32, 16), 2)

        def fire_gather(g, b):
            pltpu.async_copy(emb_hbm.at[ids_v.at[g]], emb_v[b], sem_e[b])
            pltpu.async_copy(proj_hbm.at[ids_v.at[g]], proj_v[b], sem_p[b])

        def wait_gather(g, b):
            pltpu.make_async_copy(
                emb_hbm.at[ids_v.at[g]], emb_v[b], sem_e[b]).wait()
            pltpu.make_async_copy(
                proj_hbm.at[ids_v.at[g]], proj_v[b], sem_p[b]).wait()

        def fire_store(g, b):
            pltpu.async_copy(
                out_v[b], out_hbm.at[pl.ds(base + g * CHUNK, CHUNK)], sem_o[b])

        def wait_store(g, b):
            pltpu.make_async_copy(
                out_v[b], out_hbm.at[pl.ds(base + g * CHUNK, CHUNK)],
                sem_o[b]).wait()

        def compute(b):
            ev, pv, ov = emb_v[b], proj_v[b], out_v[b]

            @plsc.parallel_loop(0, CHUNK, unroll=2)
            def _(t):
                tsplat = jnp.full((16,), t, jnp.int32)
                p0 = pv[t, pl.ds(0, 16)]
                p1 = pv[t, pl.ds(16, 16)]
                for j in range(OUT_W // LANES):
                    e = plsc.load_gather(ev, [tsplat, quarter + 4 * j])
                    p = p0 if j % 2 == 0 else p1
                    ov[t, pl.ds(LANES * j, LANES)] = e * p

        fire_gather(0, 0)
        fire_gather(1, 1)

        def pair_body(i, _):
            for b in range(2):
                g = 2 * i + b

                @pl.when(i > 0)
                def _():
                    wait_store(g - 2, b)

                wait_gather(g, b)
                compute(b)
                fire_store(g, b)

                @pl.when(g + 2 < n_chunks)
                def _():
                    fire_gather(g + 2, b)
            return 0

        lax.fori_loop(0, n_chunks // 2, pair_body, 0)
        wait_store(n_chunks - 2, 0)
        wait_store(n_chunks - 1, 1)

    return kern


def kernel(input_ids, emb_table, proj_table):
    bs, l = input_ids.shape
    num_tokens = bs * l
    ids = input_ids.reshape(-1).astype(jnp.int32)
    per_w = num_tokens // NW
    ids3 = ids.reshape(NW, per_w // CHUNK, CHUNK)
    out = _sc_kernel(num_tokens)(ids3, emb_table, proj_table)
    return out.reshape(bs, l, OUT_W)


# speedup vs baseline: 2.9832x; 1.4276x over previous
import functools

import jax
import jax.numpy as jnp
import numpy as np
from jax import lax
from jax.experimental import pallas as pl
from jax.experimental.pallas import tpu as pltpu
from jax.experimental.pallas import tpu_sc as plsc

NUM_EMB = 100000
HIDDEN = 128
PROJ_W = 32
OUT_W = 512
LANES = 16

NC, NS = 2, 16
NW = NC * NS

CHUNK = 64

_GATHER_DN = lax.GatherDimensionNumbers(
    offset_dims=(), collapsed_slice_dims=(0,), start_index_map=(0,))


def _vperm(vec, idx):
    return lax.gather(vec, idx[:, None], _GATHER_DN, (1,),
                      mode=lax.GatherScatterMode.PROMISE_IN_BOUNDS)


def _sc_kernel(num_tokens):
    assert num_tokens % (NW * CHUNK) == 0
    per_w = num_tokens // NW
    n_chunks = per_w // CHUNK
    assert n_chunks % 2 == 0
    mesh = plsc.VectorSubcoreMesh(core_axis_name="c", subcore_axis_name="s")

    @functools.partial(
        pl.kernel,
        out_type=jax.ShapeDtypeStruct((num_tokens, OUT_W), jnp.float32),
        mesh=mesh,
        compiler_params=pltpu.CompilerParams(
            use_tc_tiling_on_sc=False, needs_layout_passes=False),
        scratch_types=[
            pltpu.VMEM((n_chunks, CHUNK), jnp.int32),
            [pltpu.VMEM((CHUNK, HIDDEN), jnp.float32) for _ in range(2)],
            [pltpu.VMEM((CHUNK, PROJ_W), jnp.float32) for _ in range(2)],
            [pltpu.VMEM((CHUNK, OUT_W), jnp.float32) for _ in range(2)],
            [pltpu.SemaphoreType.DMA for _ in range(2)],
            [pltpu.SemaphoreType.DMA for _ in range(2)],
            [pltpu.SemaphoreType.DMA for _ in range(2)],
        ],
    )
    def kern(ids_hbm, emb_hbm, proj_hbm, out_hbm,
             ids_v, emb_v, proj_v, out_v, sem_e, sem_p, sem_o):
        wid = lax.axis_index("s") * NC + lax.axis_index("c")
        base = wid * per_w
        pltpu.sync_copy(ids_hbm.at[wid], ids_v)

        quarter = lax.shift_right_logical(lax.iota(jnp.int32, 16), 2)

        def fire_gather(g, b):
            pltpu.async_copy(emb_hbm.at[ids_v.at[g]], emb_v[b], sem_e[b])
            pltpu.async_copy(proj_hbm.at[ids_v.at[g]], proj_v[b], sem_p[b])

        def wait_gather(g, b):
            pltpu.make_async_copy(
                emb_hbm.at[ids_v.at[g]], emb_v[b], sem_e[b]).wait()
            pltpu.make_async_copy(
                proj_hbm.at[ids_v.at[g]], proj_v[b], sem_p[b]).wait()

        def fire_store(g, b):
            pltpu.async_copy(
                out_v[b], out_hbm.at[pl.ds(base + g * CHUNK, CHUNK)], sem_o[b])

        def wait_store(g, b):
            pltpu.make_async_copy(
                out_v[b], out_hbm.at[pl.ds(base + g * CHUNK, CHUNK)],
                sem_o[b]).wait()

        def compute(b):
            ev, pv, ov = emb_v[b], proj_v[b], out_v[b]

            @plsc.parallel_loop(0, CHUNK, unroll=2)
            def _(t):
                p0 = pv[t, pl.ds(0, 16)]
                p1 = pv[t, pl.ds(16, 16)]
                for v in range(HIDDEN // LANES):
                    evec = ev[t, pl.ds(LANES * v, LANES)]
                    for r in range(4):
                        j = 4 * v + r
                        e = _vperm(evec, quarter + 4 * r)
                        p = p0 if j % 2 == 0 else p1
                        ov[t, pl.ds(LANES * j, LANES)] = e * p

        fire_gather(0, 0)
        fire_gather(1, 1)

        def pair_body(i, _):
            for b in range(2):
                g = 2 * i + b

                @pl.when(i > 0)
                def _():
                    wait_store(g - 2, b)

                wait_gather(g, b)
                compute(b)
                fire_store(g, b)

                @pl.when(g + 2 < n_chunks)
                def _():
                    fire_gather(g + 2, b)
            return 0

        lax.fori_loop(0, n_chunks // 2, pair_body, 0)
        wait_store(n_chunks - 2, 0)
        wait_store(n_chunks - 1, 1)

    return kern


def kernel(input_ids, emb_table, proj_table):
    bs, l = input_ids.shape
    num_tokens = bs * l
    ids = input_ids.reshape(-1).astype(jnp.int32)
    per_w = num_tokens // NW
    ids3 = ids.reshape(NW, per_w // CHUNK, CHUNK)
    out = _sc_kernel(num_tokens)(ids3, emb_table, proj_table)
    return out.reshape(bs, l, OUT_W)
